# baseline (speedup 1.0000x reference)
import functools

import jax
import jax.numpy as jnp
from jax import lax
from jax.experimental import pallas as pl
from jax.experimental.pallas import tpu as pltpu
from jax.experimental.pallas import tpu_sc as plsc

N_FACTORS = 64
BATCH = 16384
LANES = 16

_info = plsc.get_sparse_core_info()
NUM_CORES = _info.num_cores
NUM_SUBCORES = _info.num_subcores
NUM_WORKERS = NUM_CORES * NUM_SUBCORES
BPW = BATCH // NUM_WORKERS

_COMPILER_PARAMS = pltpu.CompilerParams(
    needs_layout_passes=False, use_tc_tiling_on_sc=False)


def _dot_body(users, movies, uemb, memb, dots,
              idx_u, idx_m, rows_u, rows_m, out_v, sem1, sem2):
    wid = lax.axis_index("s") * NUM_CORES + lax.axis_index("c")
    base = wid * BPW

    pltpu.sync_copy(users.at[pl.ds(base, BPW)], idx_u)
    pltpu.sync_copy(movies.at[pl.ds(base, BPW)], idx_m)
    cp1 = pltpu.async_copy(uemb.at[idx_u], rows_u, sem1)
    cp2 = pltpu.async_copy(memb.at[idx_m], rows_m, sem2)
    cp1.wait()
    cp2.wait()

    lane = lax.broadcasted_iota(jnp.int32, (LANES,), 0)

    def chunk_body(c, carry):
        off = c * LANES
        rvec = off + lane
        accs = [jnp.zeros((LANES,), jnp.float32) for _ in range(4)]
        for f in range(N_FACTORS):
            col = lane ^ f
            uu = plsc.load_gather(rows_u, [rvec, col])
            mm = plsc.load_gather(rows_m, [rvec, col])
            accs[f % 4] = accs[f % 4] + uu * mm
        out_v[pl.ds(off, LANES)] = (accs[0] + accs[1]) + (accs[2] + accs[3])
        return carry

    lax.fori_loop(0, BPW // LANES, chunk_body, 0)
    pltpu.sync_copy(out_v, dots.at[pl.ds(base, BPW)])


def _bias_body(dots, users, movies, ubias, mbias, out,
               idx_u, idx_m, d_v, b_u, b_m, sem1, sem2):
    wid = lax.axis_index("s") * NUM_CORES + lax.axis_index("c")
    base = wid * BPW

    pltpu.sync_copy(users.at[pl.ds(base, BPW)], idx_u)
    pltpu.sync_copy(movies.at[pl.ds(base, BPW)], idx_m)
    cp1 = pltpu.async_copy(ubias.at[idx_u], b_u, sem1)
    cp2 = pltpu.async_copy(mbias.at[idx_m], b_m, sem2)
    pltpu.sync_copy(dots.at[pl.ds(base, BPW)], d_v)
    cp1.wait()
    cp2.wait()

    def chunk_body(c, carry):
        off = c * LANES
        d_v[pl.ds(off, LANES)] = (d_v[pl.ds(off, LANES)] +
                                  b_u[pl.ds(off, LANES)] +
                                  b_m[pl.ds(off, LANES)])
        return carry

    lax.fori_loop(0, BPW // LANES, chunk_body, 0)
    pltpu.sync_copy(d_v, out.at[pl.ds(base, BPW)])


@jax.jit
def kernel(users, movies, user_embedding, movie_embedding, user_bias,
           movie_bias):
    users = users.astype(jnp.int32)
    movies = movies.astype(jnp.int32)
    ubias = jnp.reshape(user_bias, (-1,))
    mbias = jnp.reshape(movie_bias, (-1,))

    mesh = plsc.VectorSubcoreMesh(core_axis_name="c", subcore_axis_name="s")

    dot_k = functools.partial(
        pl.kernel,
        out_type=jax.ShapeDtypeStruct((BATCH,), jnp.float32),
        mesh=mesh,
        compiler_params=_COMPILER_PARAMS,
        scratch_types=[
            pltpu.VMEM((BPW,), jnp.int32),
            pltpu.VMEM((BPW,), jnp.int32),
            pltpu.VMEM((BPW, N_FACTORS), jnp.float32),
            pltpu.VMEM((BPW, N_FACTORS), jnp.float32),
            pltpu.VMEM((BPW,), jnp.float32),
            pltpu.SemaphoreType.DMA,
            pltpu.SemaphoreType.DMA,
        ],
    )(_dot_body)
    dots = dot_k(users, movies, user_embedding, movie_embedding)

    bias_k = functools.partial(
        pl.kernel,
        out_type=jax.ShapeDtypeStruct((BATCH,), jnp.float32),
        mesh=mesh,
        compiler_params=_COMPILER_PARAMS,
        scratch_types=[
            pltpu.VMEM((BPW,), jnp.int32),
            pltpu.VMEM((BPW,), jnp.int32),
            pltpu.VMEM((BPW,), jnp.float32),
            pltpu.VMEM((BPW,), jnp.float32),
            pltpu.VMEM((BPW,), jnp.float32),
            pltpu.SemaphoreType.DMA,
            pltpu.SemaphoreType.DMA,
        ],
    )(_bias_body)
    return bias_k(dots, users, movies, ubias, mbias)


# speedup vs baseline: 1.0336x; 1.0336x over previous
import functools

import jax
import jax.numpy as jnp
from jax import lax
from jax.experimental import pallas as pl
from jax.experimental.pallas import tpu as pltpu
from jax.experimental.pallas import tpu_sc as plsc

N_FACTORS = 64
BATCH = 16384
LANES = 16
BIAS_W = 8

_info = plsc.get_sparse_core_info()
NUM_CORES = _info.num_cores
NUM_SUBCORES = _info.num_subcores
NUM_WORKERS = NUM_CORES * NUM_SUBCORES
BPW = BATCH // NUM_WORKERS


def _sc_body(users, movies, uemb, memb, ubias, mbias, out,
             idx_u, idx_m, idx_u8, idx_m8, rows_u, rows_m, b_u, b_m, out_v,
             sem1, sem2, sem3, sem4):
    wid = lax.axis_index("s") * NUM_CORES + lax.axis_index("c")
    base = wid * BPW

    pltpu.sync_copy(users.at[pl.ds(base, BPW)], idx_u)
    pltpu.sync_copy(movies.at[pl.ds(base, BPW)], idx_m)

    cp1 = pltpu.async_copy(uemb.at[idx_u], rows_u, sem1)
    cp2 = pltpu.async_copy(memb.at[idx_m], rows_m, sem2)

    def shift_body(c, carry):
        off = c * LANES
        idx_u8[pl.ds(off, LANES)] = lax.shift_right_logical(
            idx_u[pl.ds(off, LANES)], 3)
        idx_m8[pl.ds(off, LANES)] = lax.shift_right_logical(
            idx_m[pl.ds(off, LANES)], 3)
        return carry

    lax.fori_loop(0, BPW // LANES, shift_body, 0)
    cp3 = pltpu.async_copy(ubias.at[idx_u8], b_u, sem3)
    cp4 = pltpu.async_copy(mbias.at[idx_m8], b_m, sem4)
    cp1.wait()
    cp2.wait()
    cp3.wait()
    cp4.wait()

    lane = lax.broadcasted_iota(jnp.int32, (LANES,), 0)

    def chunk_body(c, carry):
        off = c * LANES
        rvec = off + lane
        col_u = idx_u[pl.ds(off, LANES)] & (BIAS_W - 1)
        col_m = idx_m[pl.ds(off, LANES)] & (BIAS_W - 1)
        acc0 = (plsc.load_gather(b_u, [rvec, col_u]) +
                plsc.load_gather(b_m, [rvec, col_m]))
        accs = [acc0] + [jnp.zeros((LANES,), jnp.float32) for _ in range(3)]
        for f in range(N_FACTORS):
            col = lane ^ f
            uu = plsc.load_gather(rows_u, [rvec, col])
            mm = plsc.load_gather(rows_m, [rvec, col])
            accs[f % 4] = accs[f % 4] + uu * mm
        out_v[pl.ds(off, LANES)] = (accs[0] + accs[1]) + (accs[2] + accs[3])
        return carry

    lax.fori_loop(0, BPW // LANES, chunk_body, 0)
    pltpu.sync_copy(out_v, out.at[pl.ds(base, BPW)])


@jax.jit
def kernel(users, movies, user_embedding, movie_embedding, user_bias,
           movie_bias):
    users = users.astype(jnp.int32)
    movies = movies.astype(jnp.int32)
    ubias = jnp.reshape(user_bias, (-1, BIAS_W))
    mbias = jnp.reshape(movie_bias, (-1, BIAS_W))

    mesh = plsc.VectorSubcoreMesh(core_axis_name="c", subcore_axis_name="s")
    run = functools.partial(
        pl.kernel,
        out_type=jax.ShapeDtypeStruct((BATCH,), jnp.float32),
        mesh=mesh,
        compiler_params=pltpu.CompilerParams(
            needs_layout_passes=False, use_tc_tiling_on_sc=False),
        scratch_types=[
            pltpu.VMEM((BPW,), jnp.int32),
            pltpu.VMEM((BPW,), jnp.int32),
            pltpu.VMEM((BPW,), jnp.int32),
            pltpu.VMEM((BPW,), jnp.int32),
            pltpu.VMEM((BPW, N_FACTORS), jnp.float32),
            pltpu.VMEM((BPW, N_FACTORS), jnp.float32),
            pltpu.VMEM((BPW, BIAS_W), jnp.float32),
            pltpu.VMEM((BPW, BIAS_W), jnp.float32),
            pltpu.VMEM((BPW,), jnp.float32),
            pltpu.SemaphoreType.DMA,
            pltpu.SemaphoreType.DMA,
            pltpu.SemaphoreType.DMA,
            pltpu.SemaphoreType.DMA,
        ],
    )(_sc_body)
    return run(users, movies, user_embedding, movie_embedding, ubias, mbias)
